# baseline (speedup 1.0000x reference)
import functools

import jax
import jax.numpy as jnp
from jax import lax
from jax.experimental import pallas as pl

K_TOPK = 64


def _to_key(bits):
    return jnp.where(bits < 0, bits ^ 0x7FFFFFFF, bits)


def _enc_body(x_ref, w_ref, be_ref, keys_ref, t_ref, *, nl, bl, bm, l_dim,
              cc, k_top):
    l = pl.program_id(1)

    @pl.when(l < nl)
    def _():
        xb = x_ref[...].astype(jnp.bfloat16)
        wb = w_ref[...].astype(jnp.bfloat16)
        zb = jnp.dot(xb, wb, preferred_element_type=jnp.float32) + be_ref[...]
        key = _to_key(lax.bitcast_convert_type(zb, jnp.int32))
        keys_ref[:, pl.ds(l * bl, bl)] = lax.bitcast_convert_type(
            key, jnp.float32)

    @pl.when(l == nl)
    def _():
        chunk = l_dim // cc

        def count(cand):
            def body(c, cnt):
                key = lax.bitcast_convert_type(
                    keys_ref[:, pl.ds(c * chunk, chunk)], jnp.int32)
                return cnt + jnp.sum((key >= cand).astype(jnp.int32),
                                     axis=1, keepdims=True)
            return lax.fori_loop(0, cc, body, jnp.zeros((bm, 1), jnp.int32))

        def step(i, t):
            cand = t + (jnp.int32(1) << (31 - i))
            return jnp.where(count(cand) >= k_top, cand, t)

        t_ref[...] = lax.fori_loop(
            0, 32, step, jnp.full((bm, 1), -(2**31), jnp.int32))


def _dec_body(keys_ref, t_ref, w_ref, bd_ref, z_ref, r_ref):
    k = pl.program_id(1)
    key = lax.bitcast_convert_type(keys_ref[...], jnp.int32)
    val = lax.bitcast_convert_type(_to_key(key), jnp.float32)
    zb = jnp.where(key >= t_ref[...], val, 0.0)
    z_ref[...] = zb
    part = jnp.dot(zb.astype(jnp.bfloat16), w_ref[...].astype(jnp.bfloat16),
                   preferred_element_type=jnp.float32)

    @pl.when(k == 0)
    def _():
        r_ref[...] = part + bd_ref[...]

    @pl.when(k > 0)
    def _():
        r_ref[...] = r_ref[...] + part


def _impl(x, w_enc, b_enc, w_dec, b_dec, interpret=False):
    b, d = x.shape
    l_dim = w_enc.shape[1]

    bm_a = min(256, b)
    bl_a = min(512, l_dim)
    nl = l_dim // bl_a
    cc = min(32, l_dim // 128)
    keys, thr = pl.pallas_call(
        functools.partial(_enc_body, nl=nl, bl=bl_a, bm=bm_a, l_dim=l_dim,
                          cc=cc, k_top=K_TOPK),
        grid=(b // bm_a, nl + 1),
        in_specs=[
            pl.BlockSpec((bm_a, d), lambda i, j: (i, 0)),
            pl.BlockSpec((d, bl_a), lambda i, j: (0, jnp.minimum(j, nl - 1))),
            pl.BlockSpec((1, bl_a), lambda i, j: (0, jnp.minimum(j, nl - 1))),
        ],
        out_specs=[
            pl.BlockSpec((bm_a, l_dim), lambda i, j: (i, 0)),
            pl.BlockSpec((bm_a, 1), lambda i, j: (i, 0)),
        ],
        out_shape=[
            jax.ShapeDtypeStruct((b, l_dim), jnp.float32),
            jax.ShapeDtypeStruct((b, 1), jnp.int32),
        ],
        interpret=interpret,
    )(x, w_enc, b_enc.reshape(1, l_dim))

    bm_c = min(1024, b)
    bk_c = min(1024, l_dim)
    z, recon = pl.pallas_call(
        _dec_body,
        grid=(b // bm_c, l_dim // bk_c),
        in_specs=[
            pl.BlockSpec((bm_c, bk_c), lambda i, j: (i, j)),
            pl.BlockSpec((bm_c, 1), lambda i, j: (i, 0)),
            pl.BlockSpec((bk_c, d), lambda i, j: (j, 0)),
            pl.BlockSpec((1, d), lambda i, j: (0, 0)),
        ],
        out_specs=[
            pl.BlockSpec((bm_c, bk_c), lambda i, j: (i, j)),
            pl.BlockSpec((bm_c, d), lambda i, j: (i, 0)),
        ],
        out_shape=[
            jax.ShapeDtypeStruct((b, l_dim), jnp.float32),
            jax.ShapeDtypeStruct((b, d), jnp.float32),
        ],
        interpret=interpret,
    )(keys, thr, w_dec, b_dec.reshape(1, d))

    return recon, z


def kernel(x, W_enc, b_enc, W_dec, b_dec):
    return _impl(x, W_enc, b_enc, W_dec, b_dec)
